# baseline (speedup 1.0000x reference)
import functools

import jax
import jax.numpy as jnp
from jax import lax
from jax.experimental import pallas as pl
from jax.experimental.pallas import tpu as pltpu
from jax.experimental.pallas import tpu_sc as plsc

N_NODES = 100000
D = 128
N2 = 8192
N1 = 2048
B = 512
N_CLASSES = 41

NC = 2
NS = 16
NW = NC * NS

_MESH = plsc.VectorSubcoreMesh(core_axis_name="c", subcore_axis_name="s")


_S_PER_W = N2 // NW
_D_PER_W = N1 // NW


@functools.partial(
    pl.kernel,
    mesh=_MESH,
    out_type=(
        jax.ShapeDtypeStruct((N2, D), jnp.float32),
        jax.ShapeDtypeStruct((N1, D), jnp.float32),
    ),
    scratch_types=[
        pltpu.VMEM((_S_PER_W,), jnp.int32),
        pltpu.VMEM((_D_PER_W,), jnp.int32),
        pltpu.VMEM((_S_PER_W,), jnp.int32),
        pltpu.VMEM((_D_PER_W,), jnp.int32),
        pltpu.VMEM((_S_PER_W, D), jnp.float32),
        pltpu.VMEM((_D_PER_W, D), jnp.float32),
        pltpu.SemaphoreType.DMA,
        pltpu.SemaphoreType.DMA,
    ],
)
def _sc_gather_l1(features_hbm, src_nodes_hbm, s2s_hbm, s2d_hbm,
                  src_out, dst_out,
                  cidx_v, didx_v, gs_v, gd_v,
                  srows_v, drows_v, sem0, sem1):
    wid = lax.axis_index("s") * NC + lax.axis_index("c")
    sbase = wid * _S_PER_W
    dbase = wid * _D_PER_W

    pltpu.sync_copy(s2s_hbm.at[pl.ds(sbase, _S_PER_W)], cidx_v)
    pltpu.sync_copy(s2d_hbm.at[pl.ds(dbase, _D_PER_W)], didx_v)

    cp0 = pltpu.async_copy(src_nodes_hbm.at[cidx_v], gs_v, sem0)
    cp1 = pltpu.async_copy(src_nodes_hbm.at[didx_v], gd_v, sem1)
    cp0.wait()
    cp1.wait()

    cp2 = pltpu.async_copy(features_hbm.at[gs_v], srows_v, sem0)
    cp3 = pltpu.async_copy(features_hbm.at[gd_v], drows_v, sem1)
    cp2.wait()
    cp3.wait()

    pltpu.sync_copy(srows_v, src_out.at[pl.ds(sbase, _S_PER_W)])
    pltpu.sync_copy(drows_v, dst_out.at[pl.ds(dbase, _D_PER_W)])


_BLK1 = 256
_GRID = N1 // _BLK1
_NCH = 8
_CH = N2 // _NCH


def _tc_fused_body(dif1_hbm, dif2_hbm, src_hbm, dst1_ref, w1t_ref, w1b_ref,
                   is2_ref, id2_ref, w2t_ref, w2b_ref, wc_ref,
                   o_ref, s1b_ref, h1_ref, dbuf0_ref, dbuf1_ref,
                   dif2_ref, src_ref, sems, sem2, sem3):
    i = pl.program_id(0)

    def issue(block, buf_ref, slot):
        for c in range(_NCH):
            pltpu.make_async_copy(
                dif1_hbm.at[pl.ds(block * _BLK1, _BLK1),
                            pl.ds(c * _CH, _CH)],
                buf_ref.at[:, pl.ds(c * _CH, _CH)],
                sems.at[slot, c],
            ).start()

    def wait(block, buf_ref, slot):
        for c in range(_NCH):
            pltpu.make_async_copy(
                dif1_hbm.at[pl.ds(block * _BLK1, _BLK1),
                            pl.ds(c * _CH, _CH)],
                buf_ref.at[:, pl.ds(c * _CH, _CH)],
                sems.at[slot, c],
            ).wait()

    @pl.when(i == 0)
    def _():
        issue(0, dbuf0_ref, 0)
        cps = pltpu.make_async_copy(src_hbm, src_ref, sem3)
        cps.start()
        issue(1, dbuf1_ref, 1)
        cps.wait()
        s1b_ref[...] = jnp.dot(src_ref[...], w1b_ref[...],
                               preferred_element_type=jnp.float32
                               ).astype(jnp.bfloat16)

    @pl.when(jnp.logical_and(i > 0, i + 1 < _GRID))
    def _():
        @pl.when(lax.rem(i + 1, 2) == 0)
        def _():
            issue(i + 1, dbuf0_ref, 0)

        @pl.when(lax.rem(i + 1, 2) == 1)
        def _():
            issue(i + 1, dbuf1_ref, 1)

    @pl.when(i == _GRID - 2)
    def _():
        pltpu.make_async_copy(dif2_hbm, dif2_ref, sem2).start()

    def consume(buf_ref, slot):
        wait(i, buf_ref, slot)
        acc = jnp.dot(dst1_ref[...], w1t_ref[...],
                      preferred_element_type=jnp.float32)
        acc = acc + jnp.dot(buf_ref[...].astype(jnp.bfloat16), s1b_ref[...],
                            preferred_element_type=jnp.float32)
        h1_ref[pl.ds(i * _BLK1, _BLK1), :] = jnp.maximum(acc, 0.0)

    @pl.when(lax.rem(i, 2) == 0)
    def _():
        consume(dbuf0_ref, 0)

    @pl.when(lax.rem(i, 2) == 1)
    def _():
        consume(dbuf1_ref, 1)

    @pl.when(i == _GRID - 1)
    def _():
        pltpu.make_async_copy(dif2_hbm, dif2_ref, sem2).wait()
        h1b = h1_ref[...].astype(jnp.bfloat16)
        col = lax.broadcasted_iota(jnp.int32, (N1, N1), 1)
        oh_s2 = (col == is2_ref[...]).astype(jnp.bfloat16)
        src2 = jnp.dot(oh_s2, h1b, preferred_element_type=jnp.float32)
        cold = lax.broadcasted_iota(jnp.int32, (B, N1), 1)
        oh_d2 = (cold == id2_ref[...]).astype(jnp.bfloat16)
        dst2 = jnp.dot(oh_d2, h1b, preferred_element_type=jnp.float32)

        agg = jnp.dot(dif2_ref[...].astype(jnp.bfloat16),
                      src2.astype(jnp.bfloat16),
                      preferred_element_type=jnp.float32)
        h = jnp.dot(dst2, w2t_ref[...], preferred_element_type=jnp.float32)
        h = h + jnp.dot(agg, w2b_ref[...], preferred_element_type=jnp.float32)
        h = jnp.maximum(h, 0.0)
        logits = jnp.dot(h, wc_ref[...], preferred_element_type=jnp.float32)
        m = jnp.max(logits, axis=-1, keepdims=True)
        e = jnp.exp(logits - m)
        o_ref[...] = e / jnp.sum(e, axis=-1, keepdims=True)


def _tc_fused(dif1, src1, dst1, w1t, w1b, dif2, is2_2d, id2_2d, w2t, w2b,
              wc):
    return pl.pallas_call(
        _tc_fused_body,
        grid=(_GRID,),
        in_specs=[
            pl.BlockSpec(memory_space=pl.ANY),
            pl.BlockSpec(memory_space=pl.ANY),
            pl.BlockSpec(memory_space=pl.ANY),
            pl.BlockSpec((_BLK1, D), lambda i: (i, 0)),
            pl.BlockSpec((D, D), lambda i: (0, 0)),
            pl.BlockSpec((D, D), lambda i: (0, 0)),
            pl.BlockSpec((N1, 1), lambda i: (0, 0)),
            pl.BlockSpec((B, 1), lambda i: (0, 0)),
            pl.BlockSpec((D, D), lambda i: (0, 0)),
            pl.BlockSpec((D, D), lambda i: (0, 0)),
            pl.BlockSpec((D, N_CLASSES), lambda i: (0, 0)),
        ],
        out_specs=pl.BlockSpec((B, N_CLASSES), lambda i: (0, 0)),
        out_shape=jax.ShapeDtypeStruct((B, N_CLASSES), jnp.float32),
        scratch_shapes=[
            pltpu.VMEM((N2, D), jnp.bfloat16),
            pltpu.VMEM((N1, D), jnp.float32),
            pltpu.VMEM((_BLK1, N2), jnp.float32),
            pltpu.VMEM((_BLK1, N2), jnp.float32),
            pltpu.VMEM((B, N1), jnp.float32),
            pltpu.VMEM((N2, D), jnp.float32),
            pltpu.SemaphoreType.DMA((2, _NCH)),
            pltpu.SemaphoreType.DMA,
            pltpu.SemaphoreType.DMA,
        ],
    )(dif1, dif2, src1, dst1, w1t, w1b, is2_2d, id2_2d, w2t, w2b, wc)


def kernel(features, src_nodes, dstsrc2src_1, dstsrc2dst_1, dif_mat_1,
           dstsrc2src_2, dstsrc2dst_2, dif_mat_2, W1, W2, Wc):
    sn = src_nodes.astype(jnp.int32)
    i_s1 = dstsrc2src_1.astype(jnp.int32)
    i_d1 = dstsrc2dst_1.astype(jnp.int32)
    i_s2 = dstsrc2src_2.astype(jnp.int32).reshape(N1, 1)
    i_d2 = dstsrc2dst_2.astype(jnp.int32).reshape(B, 1)

    src1, dst1 = _sc_gather_l1(features, sn, i_s1, i_d1)
    return _tc_fused(dif_mat_1, src1, dst1, W1[:D], W1[D:],
                     dif_mat_2, i_s2, i_d2, W2[:D], W2[D:], Wc)


# speedup vs baseline: 1.0031x; 1.0031x over previous
import functools

import jax
import jax.numpy as jnp
from jax import lax
from jax.experimental import pallas as pl
from jax.experimental.pallas import tpu as pltpu
from jax.experimental.pallas import tpu_sc as plsc

N_NODES = 100000
D = 128
N2 = 8192
N1 = 2048
B = 512
N_CLASSES = 41

NC = 2
NS = 16
NW = NC * NS

_MESH = plsc.VectorSubcoreMesh(core_axis_name="c", subcore_axis_name="s")


_S_PER_W = N2 // NW
_D_PER_W = N1 // NW


@functools.partial(
    pl.kernel,
    mesh=_MESH,
    out_type=(
        jax.ShapeDtypeStruct((N2, D), jnp.float32),
        jax.ShapeDtypeStruct((N1, D), jnp.float32),
    ),
    scratch_types=[
        pltpu.VMEM((_S_PER_W,), jnp.int32),
        pltpu.VMEM((_D_PER_W,), jnp.int32),
        pltpu.VMEM((_S_PER_W,), jnp.int32),
        pltpu.VMEM((_D_PER_W,), jnp.int32),
        pltpu.VMEM((_S_PER_W, D), jnp.float32),
        pltpu.VMEM((_D_PER_W, D), jnp.float32),
        pltpu.SemaphoreType.DMA,
        pltpu.SemaphoreType.DMA,
    ],
)
def _sc_gather_l1(features_hbm, src_nodes_hbm, s2s_hbm, s2d_hbm,
                  src_out, dst_out,
                  cidx_v, didx_v, gs_v, gd_v,
                  srows_v, drows_v, sem0, sem1):
    wid = lax.axis_index("s") * NC + lax.axis_index("c")
    sbase = wid * _S_PER_W
    dbase = wid * _D_PER_W

    pltpu.sync_copy(s2s_hbm.at[pl.ds(sbase, _S_PER_W)], cidx_v)
    pltpu.sync_copy(s2d_hbm.at[pl.ds(dbase, _D_PER_W)], didx_v)

    cp0 = pltpu.async_copy(src_nodes_hbm.at[cidx_v], gs_v, sem0)
    cp1 = pltpu.async_copy(src_nodes_hbm.at[didx_v], gd_v, sem1)
    cp0.wait()
    cp1.wait()

    cp2 = pltpu.async_copy(features_hbm.at[gs_v], srows_v, sem0)
    cp3 = pltpu.async_copy(features_hbm.at[gd_v], drows_v, sem1)
    cp2.wait()
    cp3.wait()

    pltpu.sync_copy(srows_v, src_out.at[pl.ds(sbase, _S_PER_W)])
    pltpu.sync_copy(drows_v, dst_out.at[pl.ds(dbase, _D_PER_W)])


_BLK1 = 256
_GRID = N1 // _BLK1
_NCH = 4
_CH = N2 // _NCH


def _tc_fused_body(dif1_hbm, dif2_hbm, src_hbm, dst1_ref, w1t_ref, w1b_ref,
                   is2_ref, id2_ref, w2t_ref, w2b_ref, wc_ref,
                   o_ref, s1b_ref, h1_ref, dbuf0_ref, dbuf1_ref,
                   dif2_ref, src_ref, sems, sem2, sem3):
    i = pl.program_id(0)

    def issue(block, buf_ref, slot):
        for c in range(_NCH):
            pltpu.make_async_copy(
                dif1_hbm.at[pl.ds(block * _BLK1, _BLK1),
                            pl.ds(c * _CH, _CH)],
                buf_ref.at[:, pl.ds(c * _CH, _CH)],
                sems.at[slot, c],
            ).start()

    def wait(block, buf_ref, slot):
        for c in range(_NCH):
            pltpu.make_async_copy(
                dif1_hbm.at[pl.ds(block * _BLK1, _BLK1),
                            pl.ds(c * _CH, _CH)],
                buf_ref.at[:, pl.ds(c * _CH, _CH)],
                sems.at[slot, c],
            ).wait()

    @pl.when(i == 0)
    def _():
        issue(0, dbuf0_ref, 0)
        cps = pltpu.make_async_copy(src_hbm, src_ref, sem3)
        cps.start()
        issue(1, dbuf1_ref, 1)
        cps.wait()
        s1b_ref[...] = jnp.dot(src_ref[...], w1b_ref[...],
                               preferred_element_type=jnp.float32
                               ).astype(jnp.bfloat16)

    @pl.when(jnp.logical_and(i > 0, i + 1 < _GRID))
    def _():
        @pl.when(lax.rem(i + 1, 2) == 0)
        def _():
            issue(i + 1, dbuf0_ref, 0)

        @pl.when(lax.rem(i + 1, 2) == 1)
        def _():
            issue(i + 1, dbuf1_ref, 1)

    @pl.when(i == _GRID - 2)
    def _():
        pltpu.make_async_copy(dif2_hbm, dif2_ref, sem2).start()

    def consume(buf_ref, slot):
        wait(i, buf_ref, slot)
        acc = jnp.dot(dst1_ref[...], w1t_ref[...],
                      preferred_element_type=jnp.float32)
        acc = acc + jnp.dot(buf_ref[...].astype(jnp.bfloat16), s1b_ref[...],
                            preferred_element_type=jnp.float32)
        h1_ref[pl.ds(i * _BLK1, _BLK1), :] = jnp.maximum(acc, 0.0)

    @pl.when(lax.rem(i, 2) == 0)
    def _():
        consume(dbuf0_ref, 0)

    @pl.when(lax.rem(i, 2) == 1)
    def _():
        consume(dbuf1_ref, 1)

    @pl.when(i == _GRID - 1)
    def _():
        pltpu.make_async_copy(dif2_hbm, dif2_ref, sem2).wait()
        h1b = h1_ref[...].astype(jnp.bfloat16)
        col = lax.broadcasted_iota(jnp.int32, (N1, N1), 1)
        oh_s2 = (col == is2_ref[...]).astype(jnp.bfloat16)
        src2 = jnp.dot(oh_s2, h1b, preferred_element_type=jnp.float32)
        cold = lax.broadcasted_iota(jnp.int32, (B, N1), 1)
        oh_d2 = (cold == id2_ref[...]).astype(jnp.bfloat16)
        dst2 = jnp.dot(oh_d2, h1b, preferred_element_type=jnp.float32)

        agg = jnp.dot(dif2_ref[...].astype(jnp.bfloat16),
                      src2.astype(jnp.bfloat16),
                      preferred_element_type=jnp.float32)
        h = jnp.dot(dst2, w2t_ref[...], preferred_element_type=jnp.float32)
        h = h + jnp.dot(agg, w2b_ref[...], preferred_element_type=jnp.float32)
        h = jnp.maximum(h, 0.0)
        logits = jnp.dot(h, wc_ref[...], preferred_element_type=jnp.float32)
        m = jnp.max(logits, axis=-1, keepdims=True)
        e = jnp.exp(logits - m)
        o_ref[...] = e / jnp.sum(e, axis=-1, keepdims=True)


def _tc_fused(dif1, src1, dst1, w1t, w1b, dif2, is2_2d, id2_2d, w2t, w2b,
              wc):
    return pl.pallas_call(
        _tc_fused_body,
        grid=(_GRID,),
        in_specs=[
            pl.BlockSpec(memory_space=pl.ANY),
            pl.BlockSpec(memory_space=pl.ANY),
            pl.BlockSpec(memory_space=pl.ANY),
            pl.BlockSpec((_BLK1, D), lambda i: (i, 0)),
            pl.BlockSpec((D, D), lambda i: (0, 0)),
            pl.BlockSpec((D, D), lambda i: (0, 0)),
            pl.BlockSpec((N1, 1), lambda i: (0, 0)),
            pl.BlockSpec((B, 1), lambda i: (0, 0)),
            pl.BlockSpec((D, D), lambda i: (0, 0)),
            pl.BlockSpec((D, D), lambda i: (0, 0)),
            pl.BlockSpec((D, N_CLASSES), lambda i: (0, 0)),
        ],
        out_specs=pl.BlockSpec((B, N_CLASSES), lambda i: (0, 0)),
        out_shape=jax.ShapeDtypeStruct((B, N_CLASSES), jnp.float32),
        scratch_shapes=[
            pltpu.VMEM((N2, D), jnp.bfloat16),
            pltpu.VMEM((N1, D), jnp.float32),
            pltpu.VMEM((_BLK1, N2), jnp.float32),
            pltpu.VMEM((_BLK1, N2), jnp.float32),
            pltpu.VMEM((B, N1), jnp.float32),
            pltpu.VMEM((N2, D), jnp.float32),
            pltpu.SemaphoreType.DMA((2, _NCH)),
            pltpu.SemaphoreType.DMA,
            pltpu.SemaphoreType.DMA,
        ],
    )(dif1, dif2, src1, dst1, w1t, w1b, is2_2d, id2_2d, w2t, w2b, wc)


def kernel(features, src_nodes, dstsrc2src_1, dstsrc2dst_1, dif_mat_1,
           dstsrc2src_2, dstsrc2dst_2, dif_mat_2, W1, W2, Wc):
    sn = src_nodes.astype(jnp.int32)
    i_s1 = dstsrc2src_1.astype(jnp.int32)
    i_d1 = dstsrc2dst_1.astype(jnp.int32)
    i_s2 = dstsrc2src_2.astype(jnp.int32).reshape(N1, 1)
    i_d2 = dstsrc2dst_2.astype(jnp.int32).reshape(B, 1)

    src1, dst1 = _sc_gather_l1(features, sn, i_s1, i_d1)
    return _tc_fused(dif_mat_1, src1, dst1, W1[:D], W1[D:],
                     dif_mat_2, i_s2, i_d2, W2[:D], W2[D:], Wc)


# speedup vs baseline: 1.0479x; 1.0447x over previous
import functools

import jax
import jax.numpy as jnp
from jax import lax
from jax.experimental import pallas as pl
from jax.experimental.pallas import tpu as pltpu
from jax.experimental.pallas import tpu_sc as plsc

N_NODES = 100000
D = 128
N2 = 8192
N1 = 2048
B = 512
N_CLASSES = 41

NC = 2
NS = 16
NW = NC * NS

_MESH = plsc.VectorSubcoreMesh(core_axis_name="c", subcore_axis_name="s")


_S_PER_W = N2 // NW
_D_PER_W = N1 // NW


@functools.partial(
    pl.kernel,
    mesh=_MESH,
    out_type=(
        jax.ShapeDtypeStruct((N2, D), jnp.float32),
        jax.ShapeDtypeStruct((N1, D), jnp.float32),
    ),
    scratch_types=[
        pltpu.VMEM((_S_PER_W,), jnp.int32),
        pltpu.VMEM((_D_PER_W,), jnp.int32),
        pltpu.VMEM((_S_PER_W,), jnp.int32),
        pltpu.VMEM((_D_PER_W,), jnp.int32),
        pltpu.VMEM((_S_PER_W, D), jnp.float32),
        pltpu.VMEM((_D_PER_W, D), jnp.float32),
        pltpu.SemaphoreType.DMA,
        pltpu.SemaphoreType.DMA,
    ],
)
def _sc_gather_l1(features_hbm, src_nodes_hbm, s2s_hbm, s2d_hbm,
                  src_out, dst_out,
                  cidx_v, didx_v, gs_v, gd_v,
                  srows_v, drows_v, sem0, sem1):
    wid = lax.axis_index("s") * NC + lax.axis_index("c")
    sbase = wid * _S_PER_W
    dbase = wid * _D_PER_W

    pltpu.sync_copy(s2s_hbm.at[pl.ds(sbase, _S_PER_W)], cidx_v)
    pltpu.sync_copy(s2d_hbm.at[pl.ds(dbase, _D_PER_W)], didx_v)

    cp0 = pltpu.async_copy(src_nodes_hbm.at[cidx_v], gs_v, sem0)
    cp1 = pltpu.async_copy(src_nodes_hbm.at[didx_v], gd_v, sem1)
    cp0.wait()
    cp1.wait()

    cp2 = pltpu.async_copy(features_hbm.at[gs_v], srows_v, sem0)
    cp3 = pltpu.async_copy(features_hbm.at[gd_v], drows_v, sem1)
    cp2.wait()
    cp3.wait()

    pltpu.sync_copy(srows_v, src_out.at[pl.ds(sbase, _S_PER_W)])
    pltpu.sync_copy(drows_v, dst_out.at[pl.ds(dbase, _D_PER_W)])


_BLK1 = 256
_GRID = N1 // _BLK1
_NCH = 4
_CH = N2 // _NCH


def _tc_fused_body(dif1_hbm, dif2_hbm, src_ref, dst1_ref, w1t_ref, w1b_ref,
                   is2_ref, id2_ref, w2t_ref, w2b_ref, wc_ref,
                   o_ref, s1b_ref, h1_ref, dbuf0_ref, dbuf1_ref,
                   dif2_ref, sems, sem2):
    i = pl.program_id(0)

    def issue(block, buf_ref, slot):
        for c in range(_NCH):
            pltpu.make_async_copy(
                dif1_hbm.at[pl.ds(block * _BLK1, _BLK1),
                            pl.ds(c * _CH, _CH)],
                buf_ref.at[:, pl.ds(c * _CH, _CH)],
                sems.at[slot, c],
            ).start()

    def wait(block, buf_ref, slot):
        for c in range(_NCH):
            pltpu.make_async_copy(
                dif1_hbm.at[pl.ds(block * _BLK1, _BLK1),
                            pl.ds(c * _CH, _CH)],
                buf_ref.at[:, pl.ds(c * _CH, _CH)],
                sems.at[slot, c],
            ).wait()

    @pl.when(i == 0)
    def _():
        issue(0, dbuf0_ref, 0)
        issue(1, dbuf1_ref, 1)
        s1b_ref[...] = jnp.dot(src_ref[...], w1b_ref[...],
                               preferred_element_type=jnp.float32
                               ).astype(jnp.bfloat16)

    @pl.when(jnp.logical_and(i > 0, i + 1 < _GRID))
    def _():
        @pl.when(lax.rem(i + 1, 2) == 0)
        def _():
            issue(i + 1, dbuf0_ref, 0)

        @pl.when(lax.rem(i + 1, 2) == 1)
        def _():
            issue(i + 1, dbuf1_ref, 1)

    @pl.when(i == _GRID - 2)
    def _():
        pltpu.make_async_copy(dif2_hbm, dif2_ref, sem2).start()

    def consume(buf_ref, slot):
        wait(i, buf_ref, slot)
        acc = jnp.dot(dst1_ref[...], w1t_ref[...],
                      preferred_element_type=jnp.float32)
        acc = acc + jnp.dot(buf_ref[...].astype(jnp.bfloat16), s1b_ref[...],
                            preferred_element_type=jnp.float32)
        h1_ref[pl.ds(i * _BLK1, _BLK1), :] = jnp.maximum(acc, 0.0)

    @pl.when(lax.rem(i, 2) == 0)
    def _():
        consume(dbuf0_ref, 0)

    @pl.when(lax.rem(i, 2) == 1)
    def _():
        consume(dbuf1_ref, 1)

    @pl.when(i == _GRID - 1)
    def _():
        pltpu.make_async_copy(dif2_hbm, dif2_ref, sem2).wait()
        h1b = h1_ref[...].astype(jnp.bfloat16)
        col = lax.broadcasted_iota(jnp.int32, (N1, N1), 1)
        oh_s2 = (col == is2_ref[...]).astype(jnp.bfloat16)
        src2 = jnp.dot(oh_s2, h1b, preferred_element_type=jnp.float32)
        cold = lax.broadcasted_iota(jnp.int32, (B, N1), 1)
        oh_d2 = (cold == id2_ref[...]).astype(jnp.bfloat16)
        dst2 = jnp.dot(oh_d2, h1b, preferred_element_type=jnp.float32)

        agg = jnp.dot(dif2_ref[...].astype(jnp.bfloat16),
                      src2.astype(jnp.bfloat16),
                      preferred_element_type=jnp.float32)
        h = jnp.dot(dst2, w2t_ref[...], preferred_element_type=jnp.float32)
        h = h + jnp.dot(agg, w2b_ref[...], preferred_element_type=jnp.float32)
        h = jnp.maximum(h, 0.0)
        logits = jnp.dot(h, wc_ref[...], preferred_element_type=jnp.float32)
        m = jnp.max(logits, axis=-1, keepdims=True)
        e = jnp.exp(logits - m)
        o_ref[...] = e / jnp.sum(e, axis=-1, keepdims=True)


def _tc_fused(dif1, src1, dst1, w1t, w1b, dif2, is2_2d, id2_2d, w2t, w2b,
              wc):
    return pl.pallas_call(
        _tc_fused_body,
        grid=(_GRID,),
        in_specs=[
            pl.BlockSpec(memory_space=pl.ANY),
            pl.BlockSpec(memory_space=pl.ANY),
            pl.BlockSpec((N2, D), lambda i: (0, 0)),
            pl.BlockSpec((_BLK1, D), lambda i: (i, 0)),
            pl.BlockSpec((D, D), lambda i: (0, 0)),
            pl.BlockSpec((D, D), lambda i: (0, 0)),
            pl.BlockSpec((N1, 1), lambda i: (0, 0)),
            pl.BlockSpec((B, 1), lambda i: (0, 0)),
            pl.BlockSpec((D, D), lambda i: (0, 0)),
            pl.BlockSpec((D, D), lambda i: (0, 0)),
            pl.BlockSpec((D, N_CLASSES), lambda i: (0, 0)),
        ],
        out_specs=pl.BlockSpec((B, N_CLASSES), lambda i: (0, 0)),
        out_shape=jax.ShapeDtypeStruct((B, N_CLASSES), jnp.float32),
        scratch_shapes=[
            pltpu.VMEM((N2, D), jnp.bfloat16),
            pltpu.VMEM((N1, D), jnp.float32),
            pltpu.VMEM((_BLK1, N2), jnp.float32),
            pltpu.VMEM((_BLK1, N2), jnp.float32),
            pltpu.VMEM((B, N1), jnp.float32),
            pltpu.SemaphoreType.DMA((2, _NCH)),
            pltpu.SemaphoreType.DMA,
        ],
    )(dif1, dif2, src1, dst1, w1t, w1b, is2_2d, id2_2d, w2t, w2b, wc)


def kernel(features, src_nodes, dstsrc2src_1, dstsrc2dst_1, dif_mat_1,
           dstsrc2src_2, dstsrc2dst_2, dif_mat_2, W1, W2, Wc):
    sn = src_nodes.astype(jnp.int32)
    i_s1 = dstsrc2src_1.astype(jnp.int32)
    i_d1 = dstsrc2dst_1.astype(jnp.int32)
    i_s2 = dstsrc2src_2.astype(jnp.int32).reshape(N1, 1)
    i_d2 = dstsrc2dst_2.astype(jnp.int32).reshape(B, 1)

    src1, dst1 = _sc_gather_l1(features, sn, i_s1, i_d1)
    return _tc_fused(dif_mat_1, src1, dst1, W1[:D], W1[D:],
                     dif_mat_2, i_s2, i_d2, W2[:D], W2[D:], Wc)
